# speedup vs baseline: 3.3238x; 3.3238x over previous
import functools

import jax
import jax.numpy as jnp
from jax import lax
from jax.experimental import pallas as pl
from jax.experimental.pallas import tpu as pltpu
from jax.experimental.pallas import tpu_sc as plsc

VOCAB = 100000
EMBED_DIM = 128
BATCH = 4096
HIST = 50

NUM_CORES = 2
NUM_SUBCORES = 16
NW = NUM_CORES * NUM_SUBCORES
TOTAL = BATCH * HIST
B_PER_W = TOTAL // NW
CHUNK = 128
N_CHUNKS = B_PER_W // CHUNK
NBUF = 2


def _emb_body(x_hbm, table_hbm, out_hbm,
              idx_v, buf0, buf1, g0, g1, s0, s1):
  wid = lax.axis_index("s") * NUM_CORES + lax.axis_index("c")
  base = wid * B_PER_W

  pltpu.sync_copy(x_hbm.at[wid], idx_v)

  bufs = (buf0, buf1)
  gsems = (g0, g1)
  ssems = (s0, s1)

  for b in range(NBUF):
    pltpu.async_copy(table_hbm.at[idx_v.at[b]], bufs[b], gsems[b])

  def outer(j0, carry):
    for b in range(NBUF):
      j = j0 * NBUF + b
      pltpu.make_async_copy(table_hbm.at[idx_v.at[0]], bufs[b], gsems[b]).wait()
      pltpu.async_copy(
          bufs[b], out_hbm.at[pl.ds(base + j * CHUNK, CHUNK)], ssems[b])
      pltpu.make_async_copy(
          bufs[b], out_hbm.at[pl.ds(base, CHUNK)], ssems[b]).wait()

      @pl.when(j + NBUF < N_CHUNKS)
      def _():
        pltpu.async_copy(
            table_hbm.at[idx_v.at[j + NBUF]], bufs[b], gsems[b])
    return carry

  lax.fori_loop(0, N_CHUNKS // NBUF, outer, 0)


@functools.partial(jax.jit, static_argnames=())
def kernel(x, table):
  x_flat = x.reshape(NW, N_CHUNKS, CHUNK).astype(jnp.int32)
  mesh = plsc.VectorSubcoreMesh(
      core_axis_name="c", subcore_axis_name="s",
      num_cores=NUM_CORES, num_subcores=NUM_SUBCORES)
  out = pl.kernel(
      _emb_body,
      out_type=jax.ShapeDtypeStruct((TOTAL, EMBED_DIM), jnp.float32),
      mesh=mesh,
      scratch_types=[
          pltpu.VMEM((N_CHUNKS, CHUNK), jnp.int32),
          pltpu.VMEM((CHUNK, EMBED_DIM), jnp.float32),
          pltpu.VMEM((CHUNK, EMBED_DIM), jnp.float32),
          pltpu.SemaphoreType.DMA,
          pltpu.SemaphoreType.DMA,
          pltpu.SemaphoreType.DMA,
          pltpu.SemaphoreType.DMA,
      ],
  )(x_flat, table)
  return out.reshape(BATCH, HIST, EMBED_DIM)
